# baseline (speedup 1.0000x reference)
import functools

import jax
import jax.numpy as jnp
from jax import lax
from jax.experimental import pallas as pl
from jax.experimental.pallas import tpu as pltpu
from jax.experimental.pallas import tpu_sc as plsc

NN = 10000
NP = 10240
EE = 320000
NCORE = 2
NSUB = 16
KCH = 128
EP = 327680
ROWS_PER_TILE = NP // NSUB

BN = 1024

_SC_PARAMS = pltpu.CompilerParams(use_tc_tiling_on_sc=False)


def _mesh():
    return plsc.VectorSubcoreMesh(core_axis_name="c", subcore_axis_name="s")


def _hist_body(dstr, out, dst_v, ones_v, zb_v, hsem, acc_sh):
    c = lax.axis_index("c")
    s = lax.axis_index("s")
    wid = c * NSUB + s
    pltpu.sync_copy(dstr.at[wid], dst_v)

    zeros = jnp.zeros((16,), jnp.float32)
    ones = jnp.ones((16,), jnp.float32)

    def zloop(i, _):
        zb_v[pl.ds(i * 16, 16)] = zeros
        return 0
    lax.fori_loop(0, ROWS_PER_TILE // 16, zloop, 0)

    def oloop(i, _):
        ones_v[pl.ds(i * 16, 16)] = ones
        return 0
    lax.fori_loop(0, KCH // 16, oloop, 0)

    row0 = s * ROWS_PER_TILE
    pltpu.sync_copy(zb_v, acc_sh.at[pl.ds(row0, ROWS_PER_TILE)])
    plsc.subcore_barrier()

    nch = EP // 32 // KCH

    def pair(g, _):
        pltpu.async_copy(ones_v, acc_sh.at[dst_v.at[2 * g]], hsem, add=True)
        pltpu.sync_copy(ones_v, acc_sh.at[dst_v.at[2 * g + 1]], add=True)
        pltpu.make_async_copy(ones_v, acc_sh.at[dst_v.at[2 * g]], hsem).wait()
        return 0
    lax.fori_loop(0, nch // 2, pair, 0)

    plsc.subcore_barrier()
    pltpu.sync_copy(acc_sh.at[pl.ds(row0, ROWS_PER_TILE)],
                    out.at[c].at[pl.ds(row0, ROWS_PER_TILE)])


@jax.jit
def _hist(dstr):
    return pl.kernel(
        _hist_body,
        out_type=jax.ShapeDtypeStruct((NCORE, NP), jnp.float32),
        mesh=_mesh(),
        compiler_params=_SC_PARAMS,
        scratch_types=[
            pltpu.VMEM((EP // 32 // KCH, KCH), jnp.int32),
            pltpu.VMEM((KCH,), jnp.float32),
            pltpu.VMEM((ROWS_PER_TILE,), jnp.float32),
            pltpu.SemaphoreType.DMA,
            pltpu.VMEM_SHARED((NP,), jnp.float32),
        ],
    )(dstr)


NBUF = 4


def _agg_body(colsplit, nch, srcr, dstr, zs, out, src_v, dst_v,
              rows0, rows1, rows2, rows3, zb,
              gsem0, gsem1, gsem2, gsem3, ssem0, ssem1, ssem2, ssem3, acc):
    C = acc.shape[1]
    c = lax.axis_index("c")
    s = lax.axis_index("s")
    rows = (rows0, rows1, rows2, rows3)
    gsem = (gsem0, gsem1, gsem2, gsem3)
    ssem = (ssem0, ssem1, ssem2, ssem3)

    if colsplit:
        pltpu.sync_copy(srcr.at[s], src_v)
        pltpu.sync_copy(dstr.at[s], dst_v)
        table = zs.at[c]
    else:
        wid = c * NSUB + s
        pltpu.sync_copy(srcr.at[wid], src_v)
        pltpu.sync_copy(dstr.at[wid], dst_v)
        table = zs

    zeros = jnp.zeros((16,), jnp.float32)

    def zrow(r, _):
        def zcol(j, _):
            zb[r, pl.ds(j * 16, 16)] = zeros
            return 0
        return lax.fori_loop(0, C // 16, zcol, 0)
    lax.fori_loop(0, KCH, zrow, 0)

    row0 = s * ROWS_PER_TILE
    for k in range(ROWS_PER_TILE // KCH):
        pltpu.sync_copy(zb, acc.at[pl.ds(row0 + k * KCH, KCH)])
    plsc.subcore_barrier()

    for b in range(NBUF):
        pltpu.async_copy(table.at[src_v.at[b]], rows[b], gsem[b])

    def group(g, _):
        j0 = NBUF * g
        for b in range(NBUF):
            j = j0 + b
            pltpu.make_async_copy(table.at[src_v.at[j]], rows[b],
                                  gsem[b]).wait()
            pltpu.async_copy(rows[b], acc.at[dst_v.at[j]], ssem[b], add=True)
        for b in range(NBUF):
            j = j0 + b
            nxt = j + NBUF

            @pl.when(nxt < nch)
            def _():
                pltpu.make_async_copy(rows[b], acc.at[dst_v.at[j]],
                                      ssem[b]).wait()
                pltpu.async_copy(table.at[src_v.at[nxt]], rows[b], gsem[b])
        return 0

    lax.fori_loop(0, nch // NBUF, group, 0)
    for b in range(NBUF):
        j = nch - NBUF + b
        pltpu.make_async_copy(rows[b], acc.at[dst_v.at[j]], ssem[b]).wait()
    plsc.subcore_barrier()

    pltpu.sync_copy(acc.at[pl.ds(row0, ROWS_PER_TILE)],
                    out.at[c].at[pl.ds(row0, ROWS_PER_TILE)])


@functools.partial(jax.jit, static_argnames=("C",))
def _agg_edge(srcr, dstr, zs, C):
    nch = EP // 32 // KCH
    return pl.kernel(
        functools.partial(_agg_body, False, nch),
        out_type=jax.ShapeDtypeStruct((NCORE, NP, C), jnp.float32),
        mesh=_mesh(),
        compiler_params=_SC_PARAMS,
        scratch_types=[
            pltpu.VMEM((nch, KCH), jnp.int32),
            pltpu.VMEM((nch, KCH), jnp.int32),
        ] + [pltpu.VMEM((KCH, C), jnp.float32)] * (NBUF + 1) + [
            pltpu.SemaphoreType.DMA,
        ] * (2 * NBUF) + [
            pltpu.VMEM_SHARED((NP, C), jnp.float32),
        ],
    )(srcr, dstr, zs)


@functools.partial(jax.jit, static_argnames=("CH",))
def _agg_col(srcr, dstr, zs, CH):
    nch = EP // 16 // KCH
    return pl.kernel(
        functools.partial(_agg_body, True, nch),
        out_type=jax.ShapeDtypeStruct((NCORE, NP, CH), jnp.float32),
        mesh=_mesh(),
        compiler_params=_SC_PARAMS,
        scratch_types=[
            pltpu.VMEM((nch, KCH), jnp.int32),
            pltpu.VMEM((nch, KCH), jnp.int32),
        ] + [pltpu.VMEM((KCH, CH), jnp.float32)] * (NBUF + 1) + [
            pltpu.SemaphoreType.DMA,
        ] * (2 * NBUF) + [
            pltpu.VMEM_SHARED((NP, CH), jnp.float32),
        ],
    )(srcr, dstr, zs)


def _mm1_body(x_ref, w_ref, v_ref):
    v_ref[...] = jnp.dot(x_ref[...], w_ref[...],
                         preferred_element_type=jnp.float32)


def _tc_mm1(xp, W1):
    return pl.pallas_call(
        _mm1_body,
        grid=(NP // BN,),
        in_specs=[
            pl.BlockSpec((BN, 128), lambda i: (i, 0)),
            pl.BlockSpec((128, 64), lambda i: (0, 0)),
        ],
        out_specs=pl.BlockSpec((BN, 64), lambda i: (i, 0)),
        out_shape=jax.ShapeDtypeStruct((NP, 64), jnp.float32),
    )(xp, W1)


def _head_body(hist_ref, v_ref, dinv_ref, z_ref):
    i = pl.program_id(0)
    deg = (hist_ref[0] + hist_ref[1] + 1.0)[:, None]
    rows = i * BN + lax.broadcasted_iota(jnp.int32, (BN, 1), 0)
    dinv = jnp.where(rows < NN, lax.rsqrt(deg), 0.0)
    dinv_ref[...] = dinv
    z_ref[...] = dinv * v_ref[...]


def _tc_head(hist, v1):
    return pl.pallas_call(
        _head_body,
        grid=(NP // BN,),
        in_specs=[
            pl.BlockSpec((2, BN), lambda i: (0, i)),
            pl.BlockSpec((BN, 64), lambda i: (i, 0)),
        ],
        out_specs=[
            pl.BlockSpec((BN, 1), lambda i: (i, 0)),
            pl.BlockSpec((BN, 64), lambda i: (i, 0)),
        ],
        out_shape=[
            jax.ShapeDtypeStruct((NP, 1), jnp.float32),
            jax.ShapeDtypeStruct((NP, 64), jnp.float32),
        ],
    )(hist, v1)


def _post_body(p_ref, z_ref, dinv_ref, b_ref, w_ref, o_ref):
    dinv = dinv_ref[...]
    h = jax.nn.relu(dinv * (p_ref[0] + p_ref[1] + z_ref[...]) + b_ref[...])
    o_ref[...] = dinv * jnp.dot(h, w_ref[...],
                                preferred_element_type=jnp.float32)


def _tc_post(P, z, dinv, b, W):
    C = z.shape[1]
    C2 = W.shape[1]
    return pl.pallas_call(
        _post_body,
        grid=(NP // BN,),
        in_specs=[
            pl.BlockSpec((2, BN, C), lambda i: (0, i, 0)),
            pl.BlockSpec((BN, C), lambda i: (i, 0)),
            pl.BlockSpec((BN, 1), lambda i: (i, 0)),
            pl.BlockSpec((1, C), lambda i: (0, 0)),
            pl.BlockSpec((C, C2), lambda i: (0, 0)),
        ],
        out_specs=pl.BlockSpec((BN, C2), lambda i: (i, 0)),
        out_shape=jax.ShapeDtypeStruct((NP, C2), jnp.float32),
    )(P, z, dinv, b.reshape(1, C), W)


def _pre_body(split_out, p_ref, z_ref, dinv_ref, b_ref, w_ref, o_ref):
    dinv = dinv_ref[...]
    g = dinv * (p_ref[0] + p_ref[1] + z_ref[...])
    h = jax.nn.relu(jnp.dot(g, w_ref[...],
                            preferred_element_type=jnp.float32) + b_ref[...])
    z = dinv * h
    if split_out:
        C2 = z.shape[1]
        o_ref[0] = z[:, :C2 // 2]
        o_ref[1] = z[:, C2 // 2:]
    else:
        o_ref[...] = z


def _tc_pre(P, z, dinv, b, W, split_out=False):
    C = z.shape[1]
    C2 = W.shape[1]
    if split_out:
        out_spec = pl.BlockSpec((2, BN, C2 // 2), lambda i: (0, i, 0))
        out_shape = jax.ShapeDtypeStruct((2, NP, C2 // 2), jnp.float32)
    else:
        out_spec = pl.BlockSpec((BN, C2), lambda i: (i, 0))
        out_shape = jax.ShapeDtypeStruct((NP, C2), jnp.float32)
    return pl.pallas_call(
        functools.partial(_pre_body, split_out),
        grid=(NP // BN,),
        in_specs=[
            pl.BlockSpec((2, BN, C), lambda i: (0, i, 0)),
            pl.BlockSpec((BN, C), lambda i: (i, 0)),
            pl.BlockSpec((BN, 1), lambda i: (i, 0)),
            pl.BlockSpec((1, C2), lambda i: (0, 0)),
            pl.BlockSpec((C, C2), lambda i: (0, 0)),
        ],
        out_specs=out_spec,
        out_shape=out_shape,
    )(P, z, dinv, b.reshape(1, C2), W)


def _tail_body(p_ref, z_ref, dinv_ref, w6_ref, b6_ref, wf_ref, bf_ref, o_ref):
    dinv = dinv_ref[...]
    g = dinv * jnp.concatenate(
        [p_ref[0] + z_ref[0], p_ref[1] + z_ref[1]], axis=1)
    h = jax.nn.relu(jnp.dot(g, w6_ref[...],
                            preferred_element_type=jnp.float32) + b6_ref[...])
    o_ref[...] = jnp.dot(h, wf_ref[...],
                         preferred_element_type=jnp.float32) + bf_ref[...]


def _tc_tail(P, z, dinv, W6, b6, Wf, bf):
    return pl.pallas_call(
        _tail_body,
        grid=(NP // BN,),
        in_specs=[
            pl.BlockSpec((2, BN, 32), lambda i: (0, i, 0)),
            pl.BlockSpec((2, BN, 32), lambda i: (0, i, 0)),
            pl.BlockSpec((BN, 1), lambda i: (i, 0)),
            pl.BlockSpec((64, 128), lambda i: (0, 0)),
            pl.BlockSpec((1, 128), lambda i: (0, 0)),
            pl.BlockSpec((128, 128), lambda i: (0, 0)),
            pl.BlockSpec((1, 128), lambda i: (0, 0)),
        ],
        out_specs=pl.BlockSpec((BN, 128), lambda i: (i, 0)),
        out_shape=jax.ShapeDtypeStruct((NP, 128), jnp.float32),
    )(P, z, dinv, W6, b6.reshape(1, 128), Wf, bf.reshape(1, 128))


def kernel(x, edge_index, batch, W1, b1, W2, b2, W3, b3, W4, b4, W5, b5,
           W6, b6, Wf, bf):
    src = edge_index[0]
    dst = edge_index[1]
    pad = EP - EE
    srcp = jnp.concatenate([src, jnp.full((pad,), NN, jnp.int32)])
    dstp = jnp.concatenate([dst, jnp.full((pad,), NN, jnp.int32)])
    srcr32 = srcp.reshape(32, EP // 32 // KCH, KCH)
    dstr32 = dstp.reshape(32, EP // 32 // KCH, KCH)
    srcr16 = srcp.reshape(16, EP // 16 // KCH, KCH)
    dstr16 = dstp.reshape(16, EP // 16 // KCH, KCH)
    xp = jnp.pad(x, ((0, NP - NN), (0, 0)))

    v1 = _tc_mm1(xp, W1)
    hist = _hist(dstr32)
    dinv, z1 = _tc_head(hist, v1)

    P = _agg_edge(srcr32, dstr32, z1, 64)
    z2 = _tc_post(P, z1, dinv, b1, W2)
    P = _agg_edge(srcr32, dstr32, z2, 32)
    z3 = _tc_post(P, z2, dinv, b2, W3)
    P = _agg_edge(srcr32, dstr32, z3, 16)
    z4 = _tc_post(P, z3, dinv, b3, jnp.eye(16, dtype=jnp.float32))
    P = _agg_edge(srcr32, dstr32, z4, 16)
    z5 = _tc_pre(P, z4, dinv, b4, W4)
    P = _agg_edge(srcr32, dstr32, z5, 32)
    z6 = _tc_pre(P, z5, dinv, b5, W5, split_out=True)
    P = _agg_col(srcr16, dstr16, z6, 32)
    out = _tc_tail(P, z6, dinv, W6, b6, Wf, bf)
    return out[:NN]


# speedup vs baseline: 1.0617x; 1.0617x over previous
import functools

import jax
import jax.numpy as jnp
from jax import lax
from jax.experimental import pallas as pl
from jax.experimental.pallas import tpu as pltpu
from jax.experimental.pallas import tpu_sc as plsc

NN = 10000
NP = 10240
EE = 320000
NCORE = 2
NSUB = 16
KCH = 128
EP = 327680
NCH = EP // NSUB // KCH
ROWS_PER_TILE = NP // NSUB
NBUF = 4

_SC_PARAMS = pltpu.CompilerParams(use_tc_tiling_on_sc=False)


def _mesh():
    return plsc.VectorSubcoreMesh(core_axis_name="c", subcore_axis_name="s")


def _hist_body(dstr, out, dst_v, ones_v, zb_v, hsem, acc_sh):
    c = lax.axis_index("c")
    s = lax.axis_index("s")
    wid = c * NSUB + s
    pltpu.sync_copy(dstr.at[wid], dst_v)

    zeros = jnp.zeros((16,), jnp.float32)
    ones = jnp.ones((16,), jnp.float32)

    def zloop(i, _):
        zb_v[pl.ds(i * 16, 16)] = zeros
        return 0
    lax.fori_loop(0, ROWS_PER_TILE // 16, zloop, 0)

    def oloop(i, _):
        ones_v[pl.ds(i * 16, 16)] = ones
        return 0
    lax.fori_loop(0, KCH // 16, oloop, 0)

    row0 = s * ROWS_PER_TILE
    pltpu.sync_copy(zb_v, acc_sh.at[pl.ds(row0, ROWS_PER_TILE)])
    plsc.subcore_barrier()

    nch = EP // 32 // KCH

    def pair(g, _):
        pltpu.async_copy(ones_v, acc_sh.at[dst_v.at[2 * g]], hsem, add=True)
        pltpu.sync_copy(ones_v, acc_sh.at[dst_v.at[2 * g + 1]], add=True)
        pltpu.make_async_copy(ones_v, acc_sh.at[dst_v.at[2 * g]], hsem).wait()
        return 0
    lax.fori_loop(0, nch // 2, pair, 0)

    plsc.subcore_barrier()
    pltpu.sync_copy(acc_sh.at[pl.ds(row0, ROWS_PER_TILE)],
                    out.at[c].at[pl.ds(row0, ROWS_PER_TILE)])


@jax.jit
def _hist(dstr):
    return pl.kernel(
        _hist_body,
        out_type=jax.ShapeDtypeStruct((NCORE, NP), jnp.float32),
        mesh=_mesh(),
        compiler_params=_SC_PARAMS,
        scratch_types=[
            pltpu.VMEM((EP // 32 // KCH, KCH), jnp.int32),
            pltpu.VMEM((KCH,), jnp.float32),
            pltpu.VMEM((ROWS_PER_TILE,), jnp.float32),
            pltpu.SemaphoreType.DMA,
            pltpu.VMEM_SHARED((NP,), jnp.float32),
        ],
    )(dstr)


def _agg_body(srcr, dstr, zs, zrows, out, src_v, dst_v,
              rows0, rows1, rows2, rows3,
              gsem0, gsem1, gsem2, gsem3, ssem0, ssem1, ssem2, ssem3, acc):
    c = lax.axis_index("c")
    s = lax.axis_index("s")
    rows = (rows0, rows1, rows2, rows3)
    gsem = (gsem0, gsem1, gsem2, gsem3)
    ssem = (ssem0, ssem1, ssem2, ssem3)

    pltpu.sync_copy(srcr.at[s], src_v)
    pltpu.sync_copy(dstr.at[s], dst_v)
    table = zs.at[c]

    row0 = s * ROWS_PER_TILE
    for k in range(ROWS_PER_TILE // KCH):
        pltpu.sync_copy(zrows, acc.at[pl.ds(row0 + k * KCH, KCH)])
    plsc.subcore_barrier()

    for b in range(NBUF):
        pltpu.async_copy(table.at[src_v.at[b]], rows[b], gsem[b])

    def group(g, _):
        j0 = NBUF * g
        for b in range(NBUF):
            j = j0 + b
            pltpu.make_async_copy(table.at[src_v.at[j]], rows[b],
                                  gsem[b]).wait()
            pltpu.async_copy(rows[b], acc.at[dst_v.at[j]], ssem[b], add=True)
        for b in range(NBUF):
            j = j0 + b
            nxt = j + NBUF

            @pl.when(nxt < NCH)
            def _():
                pltpu.make_async_copy(rows[b], acc.at[dst_v.at[j]],
                                      ssem[b]).wait()
                pltpu.async_copy(table.at[src_v.at[nxt]], rows[b], gsem[b])
        return 0

    lax.fori_loop(0, NCH // NBUF, group, 0)
    for b in range(NBUF):
        j = NCH - NBUF + b
        pltpu.make_async_copy(rows[b], acc.at[dst_v.at[j]], ssem[b]).wait()
    plsc.subcore_barrier()

    pltpu.sync_copy(acc.at[pl.ds(row0, ROWS_PER_TILE)],
                    out.at[c].at[pl.ds(row0, ROWS_PER_TILE)])


@functools.partial(jax.jit, static_argnames=("CH",))
def _agg_col(srcr, dstr, zs, CH):
    zrows = jnp.zeros((KCH, CH), jnp.float32)
    return pl.kernel(
        _agg_body,
        out_type=jax.ShapeDtypeStruct((NCORE, NP, CH), jnp.float32),
        mesh=_mesh(),
        compiler_params=_SC_PARAMS,
        scratch_types=[
            pltpu.VMEM((NCH, KCH), jnp.int32),
            pltpu.VMEM((NCH, KCH), jnp.int32),
        ] + [pltpu.VMEM((KCH, CH), jnp.float32)] * NBUF + [
            pltpu.SemaphoreType.DMA,
        ] * (2 * NBUF) + [
            pltpu.VMEM_SHARED((NP, CH), jnp.float32),
        ],
    )(srcr, dstr, zs, zrows)


def _mm1_body(x_ref, w_ref, v_ref):
    v_ref[...] = jnp.dot(x_ref[...], w_ref[...],
                         preferred_element_type=jnp.float32)


def _tc_mm1(xp, W1):
    return pl.pallas_call(
        _mm1_body,
        out_shape=jax.ShapeDtypeStruct((NP, 64), jnp.float32),
    )(xp, W1)


def _head_body(hist_ref, v_ref, dinv_ref, z_ref):
    deg = (hist_ref[0] + hist_ref[1] + 1.0)[:, None]
    rows = lax.broadcasted_iota(jnp.int32, (NP, 1), 0)
    dinv = jnp.where(rows < NN, lax.rsqrt(deg), 0.0)
    dinv_ref[...] = dinv
    z = dinv * v_ref[...]
    z_ref[0] = z[:, :32]
    z_ref[1] = z[:, 32:]


def _tc_head(hist, v1):
    return pl.pallas_call(
        _head_body,
        out_shape=[
            jax.ShapeDtypeStruct((NP, 1), jnp.float32),
            jax.ShapeDtypeStruct((2, NP, 32), jnp.float32),
        ],
    )(hist, v1)


def _post_body(p_ref, z_ref, dinv_ref, b_ref, w_ref, o_ref):
    dinv = dinv_ref[...]
    g = jnp.concatenate([p_ref[0] + z_ref[0], p_ref[1] + z_ref[1]], axis=1)
    h = jax.nn.relu(dinv * g + b_ref[...])
    zn = dinv * jnp.dot(h, w_ref[...], preferred_element_type=jnp.float32)
    C2 = w_ref.shape[1]
    o_ref[0] = zn[:, :C2 // 2]
    o_ref[1] = zn[:, C2 // 2:]


def _tc_post(P, z, dinv, b, W):
    C = 2 * z.shape[2]
    C2 = W.shape[1]
    return pl.pallas_call(
        _post_body,
        out_shape=jax.ShapeDtypeStruct((2, NP, C2 // 2), jnp.float32),
    )(P, z, dinv, b.reshape(1, C), W)


def _pre_body(p_ref, z_ref, dinv_ref, b_ref, w_ref, o_ref):
    dinv = dinv_ref[...]
    g = dinv * jnp.concatenate(
        [p_ref[0] + z_ref[0], p_ref[1] + z_ref[1]], axis=1)
    h = jax.nn.relu(jnp.dot(g, w_ref[...],
                            preferred_element_type=jnp.float32) + b_ref[...])
    zn = dinv * h
    C2 = w_ref.shape[1]
    o_ref[0] = zn[:, :C2 // 2]
    o_ref[1] = zn[:, C2 // 2:]


def _tc_pre(P, z, dinv, b, W):
    C2 = W.shape[1]
    return pl.pallas_call(
        _pre_body,
        out_shape=jax.ShapeDtypeStruct((2, NP, C2 // 2), jnp.float32),
    )(P, z, dinv, b.reshape(1, C2), W)


def _tail_body(p_ref, z_ref, dinv_ref, w6_ref, b6_ref, wf_ref, bf_ref, o_ref):
    dinv = dinv_ref[...]
    g = dinv * jnp.concatenate(
        [p_ref[0] + z_ref[0], p_ref[1] + z_ref[1]], axis=1)
    h = jax.nn.relu(jnp.dot(g, w6_ref[...],
                            preferred_element_type=jnp.float32) + b6_ref[...])
    o_ref[...] = jnp.dot(h, wf_ref[...],
                         preferred_element_type=jnp.float32) + bf_ref[...]


def _tc_tail(P, z, dinv, W6, b6, Wf, bf):
    return pl.pallas_call(
        _tail_body,
        out_shape=jax.ShapeDtypeStruct((NP, 128), jnp.float32),
    )(P, z, dinv, W6, b6.reshape(1, 128), Wf, bf.reshape(1, 128))


def kernel(x, edge_index, batch, W1, b1, W2, b2, W3, b3, W4, b4, W5, b5,
           W6, b6, Wf, bf):
    src = edge_index[0]
    dst = edge_index[1]
    pad = EP - EE
    srcp = jnp.concatenate([src, jnp.full((pad,), NN, jnp.int32)])
    dstp = jnp.concatenate([dst, jnp.full((pad,), NN, jnp.int32)])
    srcr = srcp.reshape(NSUB, NCH, KCH)
    dstr = dstp.reshape(NSUB, NCH, KCH)
    dstr32 = dstp.reshape(32, EP // 32 // KCH, KCH)
    xp = jnp.pad(x, ((0, NP - NN), (0, 0)))

    v1 = _tc_mm1(xp, W1)
    hist = _hist(dstr32)
    dinv, z1 = _tc_head(hist, v1)

    P = _agg_col(srcr, dstr, z1, 32)
    z2 = _tc_post(P, z1, dinv, b1, W2)
    P = _agg_col(srcr, dstr, z2, 16)
    z3 = _tc_post(P, z2, dinv, b2, W3)
    P = _agg_col(srcr, dstr, z3, 8)
    z4 = _tc_post(P, z3, dinv, b3, jnp.eye(16, dtype=jnp.float32))
    P = _agg_col(srcr, dstr, z4, 8)
    z5 = _tc_pre(P, z4, dinv, b4, W4)
    P = _agg_col(srcr, dstr, z5, 16)
    z6 = _tc_pre(P, z5, dinv, b5, W5)
    P = _agg_col(srcr, dstr, z6, 32)
    out = _tc_tail(P, z6, dinv, W6, b6, Wf, bf)
    return out[:NN]


# speedup vs baseline: 1.0994x; 1.0355x over previous
import functools

import jax
import jax.numpy as jnp
from jax import lax
from jax.experimental import pallas as pl
from jax.experimental.pallas import tpu as pltpu
from jax.experimental.pallas import tpu_sc as plsc

NN = 10000
NP = 10240
EE = 320000
NCORE = 2
NSUB = 16
KCH = 128
EP = 327680
NCH = EP // NSUB // KCH
ROWS_PER_TILE = NP // NSUB
NBUF = 4

_SC_PARAMS = pltpu.CompilerParams(use_tc_tiling_on_sc=False)


def _mesh():
    return plsc.VectorSubcoreMesh(core_axis_name="c", subcore_axis_name="s")


def _hist_body(dstr, out, dst_v, ones_v, zb_v, hsem, acc_sh):
    c = lax.axis_index("c")
    s = lax.axis_index("s")
    wid = c * NSUB + s
    pltpu.sync_copy(dstr.at[wid], dst_v)

    zeros = jnp.zeros((16,), jnp.float32)
    ones = jnp.ones((16,), jnp.float32)

    def zloop(i, _):
        zb_v[pl.ds(i * 16, 16)] = zeros
        return 0
    lax.fori_loop(0, ROWS_PER_TILE // 16, zloop, 0)

    def oloop(i, _):
        ones_v[pl.ds(i * 16, 16)] = ones
        return 0
    lax.fori_loop(0, KCH // 16, oloop, 0)

    row0 = s * ROWS_PER_TILE
    pltpu.sync_copy(zb_v, acc_sh.at[pl.ds(row0, ROWS_PER_TILE)])
    plsc.subcore_barrier()

    nch = EP // 32 // KCH

    def pair(g, _):
        pltpu.async_copy(ones_v, acc_sh.at[dst_v.at[2 * g]], hsem, add=True)
        pltpu.sync_copy(ones_v, acc_sh.at[dst_v.at[2 * g + 1]], add=True)
        pltpu.make_async_copy(ones_v, acc_sh.at[dst_v.at[2 * g]], hsem).wait()
        return 0
    lax.fori_loop(0, nch // 2, pair, 0)

    plsc.subcore_barrier()
    pltpu.sync_copy(acc_sh.at[pl.ds(row0, ROWS_PER_TILE)],
                    out.at[c].at[pl.ds(row0, ROWS_PER_TILE)])


@jax.jit
def _hist(dstr):
    return pl.kernel(
        _hist_body,
        out_type=jax.ShapeDtypeStruct((NCORE, NP), jnp.float32),
        mesh=_mesh(),
        compiler_params=_SC_PARAMS,
        scratch_types=[
            pltpu.VMEM((EP // 32 // KCH, KCH), jnp.int32),
            pltpu.VMEM((KCH,), jnp.float32),
            pltpu.VMEM((ROWS_PER_TILE,), jnp.float32),
            pltpu.SemaphoreType.DMA,
            pltpu.VMEM_SHARED((NP,), jnp.float32),
        ],
    )(dstr)


def _agg_body(nbuf, srcr, dstr, zs, zrows, out, *scratch):
    src_v, dst_v = scratch[0], scratch[1]
    rows = scratch[2:2 + nbuf]
    gsem = scratch[2 + nbuf:2 + 2 * nbuf]
    ssem = scratch[2 + 2 * nbuf:2 + 3 * nbuf]
    acc = scratch[2 + 3 * nbuf]
    NBUF = nbuf
    c = lax.axis_index("c")
    s = lax.axis_index("s")

    row0 = s * ROWS_PER_TILE
    pltpu.async_copy(srcr.at[s], src_v, gsem[0])
    pltpu.async_copy(dstr.at[s], dst_v, gsem[1])
    for k in range(ROWS_PER_TILE // KCH):
        pltpu.async_copy(zrows, acc.at[pl.ds(row0 + k * KCH, KCH)], ssem[0])
    table = zs.at[c]
    pltpu.make_async_copy(srcr.at[s], src_v, gsem[0]).wait()
    pltpu.make_async_copy(dstr.at[s], dst_v, gsem[1]).wait()
    for k in range(ROWS_PER_TILE // KCH):
        pltpu.make_async_copy(zrows, acc.at[pl.ds(row0 + k * KCH, KCH)],
                              ssem[0]).wait()
    plsc.subcore_barrier()

    for b in range(NBUF):
        pltpu.async_copy(table.at[src_v.at[b]], rows[b], gsem[b])

    def group(g, _):
        j0 = NBUF * g
        for b in range(NBUF):
            j = j0 + b
            pltpu.make_async_copy(table.at[src_v.at[j]], rows[b],
                                  gsem[b]).wait()
            pltpu.async_copy(rows[b], acc.at[dst_v.at[j]], ssem[b], add=True)
        for b in range(NBUF):
            j = j0 + b
            nxt = j + NBUF

            @pl.when(nxt < NCH)
            def _():
                pltpu.make_async_copy(rows[b], acc.at[dst_v.at[j]],
                                      ssem[b]).wait()
                pltpu.async_copy(table.at[src_v.at[nxt]], rows[b], gsem[b])
        return 0

    lax.fori_loop(0, NCH // NBUF, group, 0)
    for b in range(NBUF):
        j = NCH - NBUF + b
        pltpu.make_async_copy(rows[b], acc.at[dst_v.at[j]], ssem[b]).wait()
    plsc.subcore_barrier()

    pltpu.sync_copy(acc.at[pl.ds(row0, ROWS_PER_TILE)],
                    out.at[c].at[pl.ds(row0, ROWS_PER_TILE)])


@functools.partial(jax.jit, static_argnames=("CH", "nbuf"))
def _agg_col(srcr, dstr, zs, CH, nbuf=4):
    zrows = jnp.zeros((KCH, CH), jnp.float32)
    return pl.kernel(
        functools.partial(_agg_body, nbuf),
        out_type=jax.ShapeDtypeStruct((NCORE, NP, CH), jnp.float32),
        mesh=_mesh(),
        compiler_params=_SC_PARAMS,
        scratch_types=[
            pltpu.VMEM((NCH, KCH), jnp.int32),
            pltpu.VMEM((NCH, KCH), jnp.int32),
        ] + [pltpu.VMEM((KCH, CH), jnp.float32)] * nbuf + [
            pltpu.SemaphoreType.DMA,
        ] * (2 * nbuf) + [
            pltpu.VMEM_SHARED((NP, CH), jnp.float32),
        ],
    )(srcr, dstr, zs, zrows)


def _mm1_body(x_ref, w_ref, v_ref):
    v_ref[...] = jnp.dot(x_ref[...], w_ref[...],
                         preferred_element_type=jnp.float32)


def _tc_mm1(xp, W1):
    return pl.pallas_call(
        _mm1_body,
        out_shape=jax.ShapeDtypeStruct((NP, 64), jnp.float32),
    )(xp, W1)


def _head_body(hist_ref, v_ref, dinv_ref, z_ref):
    deg = (hist_ref[0] + hist_ref[1] + 1.0)[:, None]
    rows = lax.broadcasted_iota(jnp.int32, (NP, 1), 0)
    dinv = jnp.where(rows < NN, lax.rsqrt(deg), 0.0)
    dinv_ref[...] = dinv
    z = dinv * v_ref[...]
    z_ref[0] = z[:, :32]
    z_ref[1] = z[:, 32:]


def _tc_head(hist, v1):
    return pl.pallas_call(
        _head_body,
        out_shape=[
            jax.ShapeDtypeStruct((NP, 1), jnp.float32),
            jax.ShapeDtypeStruct((2, NP, 32), jnp.float32),
        ],
    )(hist, v1)


def _post_body(p_ref, z_ref, dinv_ref, b_ref, w_ref, o_ref):
    dinv = dinv_ref[...]
    g = jnp.concatenate([p_ref[0] + z_ref[0], p_ref[1] + z_ref[1]], axis=1)
    h = jax.nn.relu(dinv * g + b_ref[...])
    zn = dinv * jnp.dot(h, w_ref[...], preferred_element_type=jnp.float32)
    C2 = w_ref.shape[1]
    o_ref[0] = zn[:, :C2 // 2]
    o_ref[1] = zn[:, C2 // 2:]


def _tc_post(P, z, dinv, b, W):
    C = 2 * z.shape[2]
    C2 = W.shape[1]
    return pl.pallas_call(
        _post_body,
        out_shape=jax.ShapeDtypeStruct((2, NP, C2 // 2), jnp.float32),
    )(P, z, dinv, b.reshape(1, C), W)


def _pre_body(p_ref, z_ref, dinv_ref, b_ref, w_ref, o_ref):
    dinv = dinv_ref[...]
    g = dinv * jnp.concatenate(
        [p_ref[0] + z_ref[0], p_ref[1] + z_ref[1]], axis=1)
    h = jax.nn.relu(jnp.dot(g, w_ref[...],
                            preferred_element_type=jnp.float32) + b_ref[...])
    zn = dinv * h
    C2 = w_ref.shape[1]
    o_ref[0] = zn[:, :C2 // 2]
    o_ref[1] = zn[:, C2 // 2:]


def _tc_pre(P, z, dinv, b, W):
    C2 = W.shape[1]
    return pl.pallas_call(
        _pre_body,
        out_shape=jax.ShapeDtypeStruct((2, NP, C2 // 2), jnp.float32),
    )(P, z, dinv, b.reshape(1, C2), W)


def _tail_body(p_ref, z_ref, dinv_ref, w6_ref, b6_ref, wf_ref, bf_ref, o_ref):
    dinv = dinv_ref[...]
    g = dinv * jnp.concatenate(
        [p_ref[0] + z_ref[0], p_ref[1] + z_ref[1]], axis=1)
    h = jax.nn.relu(jnp.dot(g, w6_ref[...],
                            preferred_element_type=jnp.float32) + b6_ref[...])
    o_ref[...] = jnp.dot(h, wf_ref[...],
                         preferred_element_type=jnp.float32) + bf_ref[...]


def _tc_tail(P, z, dinv, W6, b6, Wf, bf):
    return pl.pallas_call(
        _tail_body,
        out_shape=jax.ShapeDtypeStruct((NP, 128), jnp.float32),
    )(P, z, dinv, W6, b6.reshape(1, 128), Wf, bf.reshape(1, 128))


def kernel(x, edge_index, batch, W1, b1, W2, b2, W3, b3, W4, b4, W5, b5,
           W6, b6, Wf, bf):
    src = edge_index[0]
    dst = edge_index[1]
    pad = EP - EE
    srcp = jnp.concatenate([src, jnp.full((pad,), NN, jnp.int32)])
    dstp = jnp.concatenate([dst, jnp.full((pad,), NN, jnp.int32)])
    srcr = srcp.reshape(NSUB, NCH, KCH)
    dstr = dstp.reshape(NSUB, NCH, KCH)
    dstr32 = dstp.reshape(32, EP // 32 // KCH, KCH)
    xp = jnp.pad(x, ((0, NP - NN), (0, 0)))

    v1 = _tc_mm1(xp, W1)
    hist = _hist(dstr32)
    dinv, z1 = _tc_head(hist, v1)

    P = _agg_col(srcr, dstr, z1, 32)
    z2 = _tc_post(P, z1, dinv, b1, W2)
    P = _agg_col(srcr, dstr, z2, 16, nbuf=8)
    z3 = _tc_post(P, z2, dinv, b2, W3)
    P = _agg_col(srcr, dstr, z3, 8, nbuf=8)
    z4 = _tc_post(P, z3, dinv, b3, jnp.eye(16, dtype=jnp.float32))
    P = _agg_col(srcr, dstr, z4, 8, nbuf=8)
    z5 = _tc_pre(P, z4, dinv, b4, W4)
    P = _agg_col(srcr, dstr, z5, 16, nbuf=8)
    z6 = _tc_pre(P, z5, dinv, b5, W5)
    P = _agg_col(srcr, dstr, z6, 32)
    out = _tc_tail(P, z6, dinv, W6, b6, Wf, bf)
    return out[:NN]


# speedup vs baseline: 1.1223x; 1.0209x over previous
import functools

import jax
import jax.numpy as jnp
from jax import lax
from jax.experimental import pallas as pl
from jax.experimental.pallas import tpu as pltpu
from jax.experimental.pallas import tpu_sc as plsc

NN = 10000
NP = 10240
EE = 320000
NCORE = 2
NSUB = 16
KCH = 128
EP = 327680
NCH = EP // NSUB // KCH
ROWS_PER_TILE = NP // NSUB
NBUF = 4

_SC_PARAMS = pltpu.CompilerParams(use_tc_tiling_on_sc=False)


def _mesh():
    return plsc.VectorSubcoreMesh(core_axis_name="c", subcore_axis_name="s")


def _hist_body(dstr, out, dst_v, ones_v, zb_v, hsem, acc_sh):
    c = lax.axis_index("c")
    s = lax.axis_index("s")
    nch = EP // 32 // KCH
    pltpu.sync_copy(dstr.at[s].at[pl.ds(c * nch, nch)], dst_v)

    zeros = jnp.zeros((16,), jnp.float32)
    ones = jnp.ones((16,), jnp.float32)

    def zloop(i, _):
        zb_v[pl.ds(i * 16, 16)] = zeros
        return 0
    lax.fori_loop(0, ROWS_PER_TILE // 16, zloop, 0)

    def oloop(i, _):
        ones_v[pl.ds(i * 16, 16)] = ones
        return 0
    lax.fori_loop(0, KCH // 16, oloop, 0)

    row0 = s * ROWS_PER_TILE
    pltpu.sync_copy(zb_v, acc_sh.at[pl.ds(row0, ROWS_PER_TILE)])
    plsc.subcore_barrier()

    nch = EP // 32 // KCH

    def pair(g, _):
        pltpu.async_copy(ones_v, acc_sh.at[dst_v.at[2 * g]], hsem, add=True)
        pltpu.sync_copy(ones_v, acc_sh.at[dst_v.at[2 * g + 1]], add=True)
        pltpu.make_async_copy(ones_v, acc_sh.at[dst_v.at[2 * g]], hsem).wait()
        return 0
    lax.fori_loop(0, nch // 2, pair, 0)

    plsc.subcore_barrier()
    pltpu.sync_copy(acc_sh.at[pl.ds(row0, ROWS_PER_TILE)],
                    out.at[c].at[pl.ds(row0, ROWS_PER_TILE)])


@jax.jit
def _hist(dstr):
    return pl.kernel(
        _hist_body,
        out_type=jax.ShapeDtypeStruct((NCORE, NP), jnp.float32),
        mesh=_mesh(),
        compiler_params=_SC_PARAMS,
        scratch_types=[
            pltpu.VMEM((EP // 32 // KCH, KCH), jnp.int32),
            pltpu.VMEM((KCH,), jnp.float32),
            pltpu.VMEM((ROWS_PER_TILE,), jnp.float32),
            pltpu.SemaphoreType.DMA,
            pltpu.VMEM_SHARED((NP,), jnp.float32),
        ],
    )(dstr)


def _agg_body(nbuf, srcr, dstr, zs, zrows, out, *scratch):
    src_v, dst_v = scratch[0], scratch[1]
    rows = scratch[2:2 + nbuf]
    gsem = scratch[2 + nbuf:2 + 2 * nbuf]
    ssem = scratch[2 + 2 * nbuf:2 + 3 * nbuf]
    acc = scratch[2 + 3 * nbuf]
    NBUF = nbuf
    c = lax.axis_index("c")
    s = lax.axis_index("s")

    row0 = s * ROWS_PER_TILE
    pltpu.async_copy(srcr.at[s], src_v, gsem[0])
    pltpu.async_copy(dstr.at[s], dst_v, gsem[1])
    for k in range(ROWS_PER_TILE // KCH):
        pltpu.async_copy(zrows, acc.at[pl.ds(row0 + k * KCH, KCH)], ssem[0])
    table = zs.at[c]
    pltpu.make_async_copy(srcr.at[s], src_v, gsem[0]).wait()
    pltpu.make_async_copy(dstr.at[s], dst_v, gsem[1]).wait()
    for k in range(ROWS_PER_TILE // KCH):
        pltpu.make_async_copy(zrows, acc.at[pl.ds(row0 + k * KCH, KCH)],
                              ssem[0]).wait()
    plsc.subcore_barrier()

    for b in range(NBUF):
        pltpu.async_copy(table.at[src_v.at[b]], rows[b], gsem[b])

    def group(g, _):
        j0 = NBUF * g
        for b in range(NBUF):
            j = j0 + b
            pltpu.make_async_copy(table.at[src_v.at[j]], rows[b],
                                  gsem[b]).wait()
            pltpu.async_copy(rows[b], acc.at[dst_v.at[j]], ssem[b], add=True)
        for b in range(NBUF):
            j = j0 + b
            nxt = j + NBUF

            @pl.when(nxt < NCH)
            def _():
                pltpu.make_async_copy(rows[b], acc.at[dst_v.at[j]],
                                      ssem[b]).wait()
                pltpu.async_copy(table.at[src_v.at[nxt]], rows[b], gsem[b])
        return 0

    lax.fori_loop(0, NCH // NBUF, group, 0)
    for b in range(NBUF):
        j = NCH - NBUF + b
        pltpu.make_async_copy(rows[b], acc.at[dst_v.at[j]], ssem[b]).wait()
    plsc.subcore_barrier()

    pltpu.sync_copy(acc.at[pl.ds(row0, ROWS_PER_TILE)],
                    out.at[c].at[pl.ds(row0, ROWS_PER_TILE)])


@functools.partial(jax.jit, static_argnames=("CH", "nbuf"))
def _agg_col(srcr, dstr, zs, CH, nbuf=4):
    zrows = jnp.zeros((KCH, CH), jnp.float32)
    return pl.kernel(
        functools.partial(_agg_body, nbuf),
        out_type=jax.ShapeDtypeStruct((NCORE, NP, CH), jnp.float32),
        mesh=_mesh(),
        compiler_params=_SC_PARAMS,
        scratch_types=[
            pltpu.VMEM((NCH, KCH), jnp.int32),
            pltpu.VMEM((NCH, KCH), jnp.int32),
        ] + [pltpu.VMEM((KCH, CH), jnp.float32)] * nbuf + [
            pltpu.SemaphoreType.DMA,
        ] * (2 * nbuf) + [
            pltpu.VMEM_SHARED((NP, CH), jnp.float32),
        ],
    )(srcr, dstr, zs, zrows)


def _head_body(hist_ref, x_ref, w_ref, dinv_ref, z_ref):
    deg = (hist_ref[0] + hist_ref[1] + 1.0)[:, None]
    rows = lax.broadcasted_iota(jnp.int32, (NP, 1), 0)
    dinv = jnp.where(rows < NN, lax.rsqrt(deg), 0.0)
    dinv_ref[...] = dinv
    z = dinv[:NN] * jnp.dot(x_ref[...], w_ref[...],
                            preferred_element_type=jnp.float32)
    z_ref[0, :NN] = z[:, :32]
    z_ref[0, NN:] = jnp.zeros((NP - NN, 32), jnp.float32)
    z_ref[1, :NN] = z[:, 32:]
    z_ref[1, NN:] = jnp.zeros((NP - NN, 32), jnp.float32)


def _tc_head(hist, x, W1):
    return pl.pallas_call(
        _head_body,
        out_shape=[
            jax.ShapeDtypeStruct((NP, 1), jnp.float32),
            jax.ShapeDtypeStruct((2, NP, 32), jnp.float32),
        ],
    )(hist, x, W1)


def _post_body(p_ref, z_ref, dinv_ref, b_ref, w_ref, o_ref):
    dinv = dinv_ref[...]
    g = jnp.concatenate([p_ref[0] + z_ref[0], p_ref[1] + z_ref[1]], axis=1)
    h = jax.nn.relu(dinv * g + b_ref[...])
    zn = dinv * jnp.dot(h, w_ref[...], preferred_element_type=jnp.float32)
    C2 = w_ref.shape[1]
    o_ref[0] = zn[:, :C2 // 2]
    o_ref[1] = zn[:, C2 // 2:]


def _tc_post(P, z, dinv, b, W):
    C = 2 * z.shape[2]
    C2 = W.shape[1]
    return pl.pallas_call(
        _post_body,
        out_shape=jax.ShapeDtypeStruct((2, NP, C2 // 2), jnp.float32),
    )(P, z, dinv, b.reshape(1, C), W)


def _pre_body(p_ref, z_ref, dinv_ref, b_ref, w_ref, o_ref):
    dinv = dinv_ref[...]
    g = dinv * jnp.concatenate(
        [p_ref[0] + z_ref[0], p_ref[1] + z_ref[1]], axis=1)
    h = jax.nn.relu(jnp.dot(g, w_ref[...],
                            preferred_element_type=jnp.float32) + b_ref[...])
    zn = dinv * h
    C2 = w_ref.shape[1]
    o_ref[0] = zn[:, :C2 // 2]
    o_ref[1] = zn[:, C2 // 2:]


def _tc_pre(P, z, dinv, b, W):
    C2 = W.shape[1]
    return pl.pallas_call(
        _pre_body,
        out_shape=jax.ShapeDtypeStruct((2, NP, C2 // 2), jnp.float32),
    )(P, z, dinv, b.reshape(1, C2), W)


def _tail_body(p_ref, z_ref, dinv_ref, w6_ref, b6_ref, wf_ref, bf_ref, o_ref):
    dinv = dinv_ref[:NN]
    g = dinv * jnp.concatenate(
        [p_ref[0, :NN] + z_ref[0, :NN], p_ref[1, :NN] + z_ref[1, :NN]],
        axis=1)
    h = jax.nn.relu(jnp.dot(g, w6_ref[...],
                            preferred_element_type=jnp.float32) + b6_ref[...])
    o_ref[...] = jnp.dot(h, wf_ref[...],
                         preferred_element_type=jnp.float32) + bf_ref[...]


def _tc_tail(P, z, dinv, W6, b6, Wf, bf):
    return pl.pallas_call(
        _tail_body,
        out_shape=jax.ShapeDtypeStruct((NN, 128), jnp.float32),
    )(P, z, dinv, W6, b6.reshape(1, 128), Wf, bf.reshape(1, 128))


def kernel(x, edge_index, batch, W1, b1, W2, b2, W3, b3, W4, b4, W5, b5,
           W6, b6, Wf, bf):
    src = edge_index[0]
    dst = edge_index[1]
    pad = EP - EE
    srcp = jnp.concatenate([src, jnp.full((pad,), NN, jnp.int32)])
    dstp = jnp.concatenate([dst, jnp.full((pad,), NN, jnp.int32)])
    srcr = srcp.reshape(NSUB, NCH, KCH)
    dstr = dstp.reshape(NSUB, NCH, KCH)

    hist = _hist(dstr)
    dinv, z1 = _tc_head(hist, x, W1)

    P = _agg_col(srcr, dstr, z1, 32)
    z2 = _tc_post(P, z1, dinv, b1, W2)
    P = _agg_col(srcr, dstr, z2, 16, nbuf=8)
    z3 = _tc_post(P, z2, dinv, b2, W3)
    P = _agg_col(srcr, dstr, z3, 8, nbuf=8)
    z4 = _tc_post(P, z3, dinv, b3, jnp.eye(16, dtype=jnp.float32))
    P = _agg_col(srcr, dstr, z4, 8, nbuf=8)
    z5 = _tc_pre(P, z4, dinv, b4, W4)
    P = _agg_col(srcr, dstr, z5, 16, nbuf=8)
    z6 = _tc_pre(P, z5, dinv, b5, W5)
    P = _agg_col(srcr, dstr, z6, 32)
    return _tc_tail(P, z6, dinv, W6, b6, Wf, bf)


# speedup vs baseline: 1.1366x; 1.0127x over previous
import functools

import jax
import jax.numpy as jnp
from jax import lax
from jax.experimental import pallas as pl
from jax.experimental.pallas import tpu as pltpu
from jax.experimental.pallas import tpu_sc as plsc

NN = 10000
NP = 10240
EE = 320000
NCORE = 2
NSUB = 16
KCH = 128
EP = 327680
NCH = EP // NSUB // KCH
ROWS_PER_TILE = NP // NSUB
NBUF = 4

_SC_PARAMS = pltpu.CompilerParams(use_tc_tiling_on_sc=False)


def _mesh():
    return plsc.VectorSubcoreMesh(core_axis_name="c", subcore_axis_name="s")


def _hist_body(dstr, out, dst_v, ones_v, zb_v, hsem, acc_sh):
    c = lax.axis_index("c")
    s = lax.axis_index("s")
    nch = EP // 32 // KCH
    pltpu.sync_copy(dstr.at[s].at[pl.ds(c * nch, nch)], dst_v)

    zeros = jnp.zeros((16,), jnp.float32)
    ones = jnp.ones((16,), jnp.float32)

    def zloop(i, _):
        zb_v[pl.ds(i * 16, 16)] = zeros
        return 0
    lax.fori_loop(0, ROWS_PER_TILE // 16, zloop, 0)

    def oloop(i, _):
        ones_v[pl.ds(i * 16, 16)] = ones
        return 0
    lax.fori_loop(0, KCH // 16, oloop, 0)

    row0 = s * ROWS_PER_TILE
    pltpu.sync_copy(zb_v, acc_sh.at[pl.ds(row0, ROWS_PER_TILE)])
    plsc.subcore_barrier()

    nch = EP // 32 // KCH

    def pair(g, _):
        pltpu.async_copy(ones_v, acc_sh.at[dst_v.at[2 * g]], hsem, add=True)
        pltpu.sync_copy(ones_v, acc_sh.at[dst_v.at[2 * g + 1]], add=True)
        pltpu.make_async_copy(ones_v, acc_sh.at[dst_v.at[2 * g]], hsem).wait()
        return 0
    lax.fori_loop(0, nch // 2, pair, 0)

    plsc.subcore_barrier()
    pltpu.sync_copy(acc_sh.at[pl.ds(row0, ROWS_PER_TILE)],
                    out.at[c].at[pl.ds(row0, ROWS_PER_TILE)])


@jax.jit
def _hist(dstr):
    return pl.kernel(
        _hist_body,
        out_type=jax.ShapeDtypeStruct((NCORE, NP), jnp.float32),
        mesh=_mesh(),
        compiler_params=_SC_PARAMS,
        scratch_types=[
            pltpu.VMEM((EP // 32 // KCH, KCH), jnp.int32),
            pltpu.VMEM((KCH,), jnp.float32),
            pltpu.VMEM((ROWS_PER_TILE,), jnp.float32),
            pltpu.SemaphoreType.DMA,
            pltpu.VMEM_SHARED((NP,), jnp.float32),
        ],
    )(dstr)


def _agg_body(nbuf, srcr, dstr, zs, zrows, out, *scratch):
    src_v, dst_v = scratch[0], scratch[1]
    rows = scratch[2:2 + nbuf]
    gsem = scratch[2 + nbuf:2 + 2 * nbuf]
    ssem = scratch[2 + 2 * nbuf:2 + 3 * nbuf]
    acc = scratch[2 + 3 * nbuf]
    NBUF = nbuf
    c = lax.axis_index("c")
    s = lax.axis_index("s")

    row0 = s * ROWS_PER_TILE
    pltpu.async_copy(srcr.at[s], src_v, gsem[0])
    pltpu.async_copy(dstr.at[s], dst_v, gsem[1])
    for k in range(ROWS_PER_TILE // KCH):
        pltpu.async_copy(zrows, acc.at[pl.ds(row0 + k * KCH, KCH)], ssem[0])
    table = zs.at[c]
    pltpu.make_async_copy(srcr.at[s], src_v, gsem[0]).wait()
    pltpu.make_async_copy(dstr.at[s], dst_v, gsem[1]).wait()
    for k in range(ROWS_PER_TILE // KCH):
        pltpu.make_async_copy(zrows, acc.at[pl.ds(row0 + k * KCH, KCH)],
                              ssem[0]).wait()
    plsc.subcore_barrier()

    for b in range(NBUF):
        pltpu.async_copy(table.at[src_v.at[b]], rows[b], gsem[b])

    def group(g, _):
        j0 = NBUF * g
        for b in range(NBUF):
            j = j0 + b
            pltpu.make_async_copy(table.at[src_v.at[j]], rows[b],
                                  gsem[b]).wait()
            pltpu.async_copy(rows[b], acc.at[dst_v.at[j]], ssem[b], add=True)
        for b in range(NBUF):
            j = j0 + b
            nxt = j + NBUF

            @pl.when(nxt < NCH)
            def _():
                pltpu.make_async_copy(rows[b], acc.at[dst_v.at[j]],
                                      ssem[b]).wait()
                pltpu.async_copy(table.at[src_v.at[nxt]], rows[b], gsem[b])
        return 0

    lax.fori_loop(0, NCH // NBUF, group, 0)
    for b in range(NBUF):
        j = NCH - NBUF + b
        pltpu.make_async_copy(rows[b], acc.at[dst_v.at[j]], ssem[b]).wait()
    plsc.subcore_barrier()

    pltpu.sync_copy(acc.at[pl.ds(row0, ROWS_PER_TILE)],
                    out.at[c].at[pl.ds(row0, ROWS_PER_TILE)])


@functools.partial(jax.jit, static_argnames=("CH", "nbuf"))
def _agg_col(srcr, dstr, zs, CH, nbuf=4):
    zrows = jnp.zeros((KCH, CH), jnp.float32)
    return pl.kernel(
        functools.partial(_agg_body, nbuf),
        out_type=jax.ShapeDtypeStruct((NCORE, NP, CH), jnp.float32),
        mesh=_mesh(),
        compiler_params=_SC_PARAMS,
        scratch_types=[
            pltpu.VMEM((NCH, KCH), jnp.int32),
            pltpu.VMEM((NCH, KCH), jnp.int32),
        ] + [pltpu.VMEM((KCH, CH), jnp.float32)] * nbuf + [
            pltpu.SemaphoreType.DMA,
        ] * (2 * nbuf) + [
            pltpu.VMEM_SHARED((NP, CH), jnp.float32),
        ],
    )(srcr, dstr, zs, zrows)


def _head_body(hist_ref, x_ref, w_ref, dinv_ref, z_ref):
    deg = (hist_ref[0] + hist_ref[1] + 1.0)[:, None]
    rows = lax.broadcasted_iota(jnp.int32, (NP, 1), 0)
    dinv = jnp.where(rows < NN, lax.rsqrt(deg), 0.0)
    dinv_ref[...] = dinv
    z = dinv[:NN] * jnp.dot(x_ref[...], w_ref[...],
                            preferred_element_type=jnp.float32)
    z_ref[0, :NN] = z[:, :32]
    z_ref[0, NN:] = jnp.zeros((NP - NN, 32), jnp.float32)
    z_ref[1, :NN] = z[:, 32:]
    z_ref[1, NN:] = jnp.zeros((NP - NN, 32), jnp.float32)


def _tc_head(hist, x, W1):
    return pl.pallas_call(
        _head_body,
        out_shape=[
            jax.ShapeDtypeStruct((NP, 1), jnp.float32),
            jax.ShapeDtypeStruct((2, NP, 32), jnp.float32),
        ],
    )(hist, x, W1)


def _post_body(p_ref, z_ref, dinv_ref, b_ref, w_ref, o_ref):
    dinv = dinv_ref[...]
    g = jnp.concatenate([p_ref[0] + z_ref[0], p_ref[1] + z_ref[1]], axis=1)
    h = jax.nn.relu(dinv * g + b_ref[...])
    zn = dinv * jnp.dot(h, w_ref[...], preferred_element_type=jnp.float32)
    C2 = w_ref.shape[1]
    o_ref[0] = zn[:, :C2 // 2]
    o_ref[1] = zn[:, C2 // 2:]


def _tc_post(P, z, dinv, b, W):
    C = 2 * z.shape[2]
    C2 = W.shape[1]
    return pl.pallas_call(
        _post_body,
        out_shape=jax.ShapeDtypeStruct((2, NP, C2 // 2), jnp.float32),
    )(P, z, dinv, b.reshape(1, C), W)


def _pre_body(p_ref, z_ref, dinv_ref, b_ref, w_ref, o_ref):
    dinv = dinv_ref[...]
    g = dinv * jnp.concatenate(
        [p_ref[0] + z_ref[0], p_ref[1] + z_ref[1]], axis=1)
    h = jax.nn.relu(jnp.dot(g, w_ref[...],
                            preferred_element_type=jnp.float32) + b_ref[...])
    zn = dinv * h
    C2 = w_ref.shape[1]
    o_ref[0] = zn[:, :C2 // 2]
    o_ref[1] = zn[:, C2 // 2:]


def _tc_pre(P, z, dinv, b, W):
    C2 = W.shape[1]
    return pl.pallas_call(
        _pre_body,
        out_shape=jax.ShapeDtypeStruct((2, NP, C2 // 2), jnp.float32),
    )(P, z, dinv, b.reshape(1, C2), W)


def _tail_body(p_ref, z_ref, dinv_ref, w6_ref, b6_ref, wf_ref, bf_ref, o_ref):
    dinv = dinv_ref[:NN]
    g = dinv * jnp.concatenate(
        [p_ref[0, :NN] + z_ref[0, :NN], p_ref[1, :NN] + z_ref[1, :NN]],
        axis=1)
    h = jax.nn.relu(jnp.dot(g, w6_ref[...],
                            preferred_element_type=jnp.float32) + b6_ref[...])
    o_ref[...] = jnp.dot(h, wf_ref[...],
                         preferred_element_type=jnp.float32) + bf_ref[...]


def _tc_tail(P, z, dinv, W6, b6, Wf, bf):
    return pl.pallas_call(
        _tail_body,
        out_shape=jax.ShapeDtypeStruct((NN, 128), jnp.float32),
    )(P, z, dinv, W6, b6.reshape(1, 128), Wf, bf.reshape(1, 128))


def kernel(x, edge_index, batch, W1, b1, W2, b2, W3, b3, W4, b4, W5, b5,
           W6, b6, Wf, bf):
    src = edge_index[0]
    dst = edge_index[1]
    pad = EP - EE
    srcp = jnp.concatenate([src, jnp.full((pad,), NN, jnp.int32)])
    dstp = jnp.concatenate([dst, jnp.full((pad,), NN, jnp.int32)])
    srcr = srcp.reshape(NSUB, NCH, KCH)
    dstr = dstp.reshape(NSUB, NCH, KCH)

    hist = _hist(dstr)
    dinv, z1 = _tc_head(hist, x, W1)

    P = _agg_col(srcr, dstr, z1, 32, nbuf=8)
    z2 = _tc_post(P, z1, dinv, b1, W2)
    P = _agg_col(srcr, dstr, z2, 16, nbuf=8)
    z3 = _tc_post(P, z2, dinv, b2, W3)
    P = _agg_col(srcr, dstr, z3, 8, nbuf=8)
    z4 = _tc_post(P, z3, dinv, b3, jnp.eye(16, dtype=jnp.float32))
    P = _agg_col(srcr, dstr, z4, 8, nbuf=8)
    z5 = _tc_pre(P, z4, dinv, b4, W4)
    P = _agg_col(srcr, dstr, z5, 16, nbuf=8)
    z6 = _tc_pre(P, z5, dinv, b5, W5)
    P = _agg_col(srcr, dstr, z6, 32, nbuf=8)
    return _tc_tail(P, z6, dinv, W6, b6, Wf, bf)


# speedup vs baseline: 1.1369x; 1.0003x over previous
import functools

import jax
import jax.numpy as jnp
from jax import lax
from jax.experimental import pallas as pl
from jax.experimental.pallas import tpu as pltpu
from jax.experimental.pallas import tpu_sc as plsc

NN = 10000
NP = 10240
EE = 320000
NCORE = 2
NSUB = 16
KCH = 128
EP = 327680
NCH = EP // NSUB // KCH
ROWS_PER_TILE = NP // NSUB

_SC_PARAMS = pltpu.CompilerParams(use_tc_tiling_on_sc=False)


def _mesh():
    return plsc.VectorSubcoreMesh(core_axis_name="c", subcore_axis_name="s")


def _hist_body(dstr, out, dst_v, ones_v, zb_v, hsem, acc_sh):
    c = lax.axis_index("c")
    s = lax.axis_index("s")
    nch = EP // 32 // KCH
    pltpu.sync_copy(dstr.at[s].at[pl.ds(c * nch, nch)], dst_v)

    zeros = jnp.zeros((16,), jnp.float32)
    ones = jnp.ones((16,), jnp.float32)

    def zloop(i, _):
        zb_v[pl.ds(i * 16, 16)] = zeros
        return 0
    lax.fori_loop(0, ROWS_PER_TILE // 16, zloop, 0)

    def oloop(i, _):
        ones_v[pl.ds(i * 16, 16)] = ones
        return 0
    lax.fori_loop(0, KCH // 16, oloop, 0)

    row0 = s * ROWS_PER_TILE
    pltpu.sync_copy(zb_v, acc_sh.at[pl.ds(row0, ROWS_PER_TILE)])
    plsc.subcore_barrier()

    def pair(g, _):
        pltpu.async_copy(ones_v, acc_sh.at[dst_v.at[2 * g]], hsem, add=True)
        pltpu.sync_copy(ones_v, acc_sh.at[dst_v.at[2 * g + 1]], add=True)
        pltpu.make_async_copy(ones_v, acc_sh.at[dst_v.at[2 * g]], hsem).wait()
        return 0
    lax.fori_loop(0, nch // 2, pair, 0)

    plsc.subcore_barrier()
    pltpu.sync_copy(acc_sh.at[pl.ds(row0, ROWS_PER_TILE)],
                    out.at[c].at[pl.ds(row0, ROWS_PER_TILE)])


@jax.jit
def _hist(dstr):
    return pl.kernel(
        _hist_body,
        out_type=jax.ShapeDtypeStruct((NCORE, NP), jnp.float32),
        mesh=_mesh(),
        compiler_params=_SC_PARAMS,
        scratch_types=[
            pltpu.VMEM((EP // 32 // KCH, KCH), jnp.int32),
            pltpu.VMEM((KCH,), jnp.float32),
            pltpu.VMEM((ROWS_PER_TILE,), jnp.float32),
            pltpu.SemaphoreType.DMA,
            pltpu.VMEM_SHARED((NP,), jnp.float32),
        ],
    )(dstr)


def _agg_body(nbuf, srcr, dstr, zs, zrows, out, *scratch):
    src_v, dst_v = scratch[0], scratch[1]
    rows = scratch[2:2 + nbuf]
    gsem = scratch[2 + nbuf:2 + 2 * nbuf]
    ssem = scratch[2 + 2 * nbuf:2 + 3 * nbuf]
    acc = scratch[2 + 3 * nbuf]
    NBUF = nbuf
    c = lax.axis_index("c")
    s = lax.axis_index("s")

    row0 = s * ROWS_PER_TILE
    pltpu.async_copy(srcr.at[s], src_v, gsem[0])
    pltpu.async_copy(dstr.at[s], dst_v, gsem[1])
    for k in range(ROWS_PER_TILE // KCH):
        pltpu.async_copy(zrows, acc.at[pl.ds(row0 + k * KCH, KCH)], ssem[0])
    table = zs.at[c]
    pltpu.make_async_copy(srcr.at[s], src_v, gsem[0]).wait()
    pltpu.make_async_copy(dstr.at[s], dst_v, gsem[1]).wait()
    for k in range(ROWS_PER_TILE // KCH):
        pltpu.make_async_copy(zrows, acc.at[pl.ds(row0 + k * KCH, KCH)],
                              ssem[0]).wait()
    plsc.subcore_barrier()

    for b in range(NBUF):
        pltpu.async_copy(table.at[src_v.at[b]], rows[b], gsem[b])

    def group(g, _):
        j0 = NBUF * g
        for b in range(NBUF):
            j = j0 + b
            pltpu.make_async_copy(table.at[src_v.at[j]], rows[b],
                                  gsem[b]).wait()
            pltpu.async_copy(rows[b], acc.at[dst_v.at[j]], ssem[b], add=True)
        for b in range(NBUF):
            j = j0 + b
            nxt = j + NBUF

            @pl.when(nxt < NCH)
            def _():
                pltpu.make_async_copy(rows[b], acc.at[dst_v.at[j]],
                                      ssem[b]).wait()
                pltpu.async_copy(table.at[src_v.at[nxt]], rows[b], gsem[b])
        return 0

    lax.fori_loop(0, NCH // NBUF, group, 0)
    for b in range(NBUF):
        j = NCH - NBUF + b
        pltpu.make_async_copy(rows[b], acc.at[dst_v.at[j]], ssem[b]).wait()
    plsc.subcore_barrier()

    pltpu.sync_copy(acc.at[pl.ds(row0, ROWS_PER_TILE)],
                    out.at[c].at[pl.ds(row0, ROWS_PER_TILE)])


@functools.partial(jax.jit, static_argnames=("CH", "nbuf"))
def _agg_col(srcr, dstr, zs, CH, nbuf=4):
    zrows = jnp.zeros((KCH, CH), jnp.float32)
    return pl.kernel(
        functools.partial(_agg_body, nbuf),
        out_type=jax.ShapeDtypeStruct((NCORE, NP, CH), jnp.float32),
        mesh=_mesh(),
        compiler_params=_SC_PARAMS,
        scratch_types=[
            pltpu.VMEM((NCH, KCH), jnp.int32),
            pltpu.VMEM((NCH, KCH), jnp.int32),
        ] + [pltpu.VMEM((KCH, CH), jnp.float32)] * nbuf + [
            pltpu.SemaphoreType.DMA,
        ] * (2 * nbuf) + [
            pltpu.VMEM_SHARED((NP, CH), jnp.float32),
        ],
    )(srcr, dstr, zs, zrows)


def _head_body(hist_ref, x_ref, w_ref, dinv_ref, z_ref):
    deg = (hist_ref[0] + hist_ref[1] + 1.0)[:, None]
    rows = lax.broadcasted_iota(jnp.int32, (NP, 1), 0)
    dinv = jnp.where(rows < NN, lax.rsqrt(deg), 0.0)
    dinv_ref[...] = dinv
    z = dinv[:NN] * jnp.dot(x_ref[...], w_ref[...],
                            preferred_element_type=jnp.float32)
    z_ref[0, :NN] = z[:, :32]
    z_ref[0, NN:] = jnp.zeros((NP - NN, 32), jnp.float32)
    z_ref[1, :NN] = z[:, 32:]
    z_ref[1, NN:] = jnp.zeros((NP - NN, 32), jnp.float32)


def _tc_head(hist, x, W1):
    return pl.pallas_call(
        _head_body,
        out_shape=[
            jax.ShapeDtypeStruct((NP, 1), jnp.float32),
            jax.ShapeDtypeStruct((2, NP, 32), jnp.float32),
        ],
    )(hist, x, W1)


def _post_body(p_ref, z_ref, dinv_ref, b_ref, w_ref, o_ref):
    dinv = dinv_ref[...]
    g = jnp.concatenate([p_ref[0] + z_ref[0], p_ref[1] + z_ref[1]], axis=1)
    h = jax.nn.relu(dinv * g + b_ref[...])
    zn = dinv * jnp.dot(h, w_ref[...], preferred_element_type=jnp.float32)
    C2 = w_ref.shape[1]
    o_ref[0] = zn[:, :C2 // 2]
    o_ref[1] = zn[:, C2 // 2:]


def _tc_post(P, z, dinv, b, W):
    C = 2 * z.shape[2]
    C2 = W.shape[1]
    return pl.pallas_call(
        _post_body,
        out_shape=jax.ShapeDtypeStruct((2, NP, C2 // 2), jnp.float32),
    )(P, z, dinv, b.reshape(1, C), W)


def _pre_body(p_ref, z_ref, dinv_ref, b_ref, w_ref, o_ref):
    dinv = dinv_ref[...]
    g = dinv * jnp.concatenate(
        [p_ref[0] + z_ref[0], p_ref[1] + z_ref[1]], axis=1)
    h = jax.nn.relu(jnp.dot(g, w_ref[...],
                            preferred_element_type=jnp.float32) + b_ref[...])
    zn = dinv * h
    C2 = w_ref.shape[1]
    o_ref[0] = zn[:, :C2 // 2]
    o_ref[1] = zn[:, C2 // 2:]


def _tc_pre(P, z, dinv, b, W):
    C2 = W.shape[1]
    return pl.pallas_call(
        _pre_body,
        out_shape=jax.ShapeDtypeStruct((2, NP, C2 // 2), jnp.float32),
    )(P, z, dinv, b.reshape(1, C2), W)


def _tail_body(p_ref, z_ref, dinv_ref, w6_ref, b6_ref, wf_ref, bf_ref, o_ref):
    dinv = dinv_ref[:NN]
    g = dinv * jnp.concatenate(
        [p_ref[0, :NN] + z_ref[0, :NN], p_ref[1, :NN] + z_ref[1, :NN]],
        axis=1)
    h = jax.nn.relu(jnp.dot(g, w6_ref[...],
                            preferred_element_type=jnp.float32) + b6_ref[...])
    o_ref[...] = jnp.dot(h, wf_ref[...],
                         preferred_element_type=jnp.float32) + bf_ref[...]


def _tc_tail(P, z, dinv, W6, b6, Wf, bf):
    return pl.pallas_call(
        _tail_body,
        out_shape=jax.ShapeDtypeStruct((NN, 128), jnp.float32),
    )(P, z, dinv, W6, b6.reshape(1, 128), Wf, bf.reshape(1, 128))


def kernel(x, edge_index, batch, W1, b1, W2, b2, W3, b3, W4, b4, W5, b5,
           W6, b6, Wf, bf):
    src = edge_index[0]
    dst = edge_index[1]
    pad = EP - EE
    srcp = jnp.concatenate([src, jnp.full((pad,), NN, jnp.int32)])
    dstp = jnp.concatenate([dst, jnp.full((pad,), NN, jnp.int32)])
    srcr = srcp.reshape(NSUB, NCH, KCH)
    dstr = dstp.reshape(NSUB, NCH, KCH)

    hist = _hist(dstr)
    dinv, z1 = _tc_head(hist, x, W1)

    P = _agg_col(srcr, dstr, z1, 32, nbuf=8)
    z2 = _tc_post(P, z1, dinv, b1, W2)
    P = _agg_col(srcr, dstr, z2, 16, nbuf=8)
    z3 = _tc_post(P, z2, dinv, b2, W3)
    P = _agg_col(srcr, dstr, z3, 8, nbuf=8)
    z4 = _tc_post(P, z3, dinv, b3, jnp.eye(16, dtype=jnp.float32))
    P = _agg_col(srcr, dstr, z4, 8, nbuf=8)
    z5 = _tc_pre(P, z4, dinv, b4, W4)
    P = _agg_col(srcr, dstr, z5, 16, nbuf=8)
    z6 = _tc_pre(P, z5, dinv, b5, W5)
    P = _agg_col(srcr, dstr, z6, 32, nbuf=8)
    return _tc_tail(P, z6, dinv, W6, b6, Wf, bf)
